# baseline (speedup 1.0000x reference)
import jax
import jax.numpy as jnp
from jax import lax
from jax.experimental import pallas as pl
from jax.experimental.pallas import tpu as pltpu
from jax.experimental.pallas import tpu_sc as plsc

B = 16384
DEG = 32
D = 128
NC = 2
NS = 16
NW = NC * NS
TPW = B // NW
ROWS_PER_DMA = 64
CT = ROWS_PER_DMA // DEG
NCHUNK = TPW // CT
NBUF = 8
NBLK = NCHUNK // NBUF
LANES = 16
GROUPS = D // LANES


def _body(neigh_hbm, table_hbm, out_hbm,
          idx_v, rows_v, outc_v, gsems, osems):
    wid = lax.axis_index("s") * NC + lax.axis_index("c")

    pltpu.sync_copy(neigh_hbm.at[wid], idx_v)

    def start_gather(chunk, b):
        pltpu.async_copy(table_hbm.at[idx_v.at[chunk]], rows_v.at[b],
                         gsems.at[b])

    def wait_gather(chunk, b):
        pltpu.make_async_copy(table_hbm.at[idx_v.at[chunk]], rows_v.at[b],
                              gsems.at[b]).wait()

    def out_slice(chunk):
        return out_hbm.at[pl.ds(wid * TPW + chunk * CT, CT)]

    def start_out(chunk, b):
        pltpu.async_copy(outc_v.at[b], out_slice(chunk), osems.at[b])

    def wait_out(chunk, b):
        pltpu.make_async_copy(outc_v.at[b], out_slice(chunk),
                              osems.at[b]).wait()

    def compute(chunk, b):
        def flush(p):
            t, sl0, sl1, acc0, acc1 = p
            outc_v[b, t, sl0] = (
                (acc0[0] + acc0[1]) + (acc0[2] + acc0[3])
            ) * (1.0 / DEG)
            outc_v[b, t, sl1] = (
                (acc1[0] + acc1[1]) + (acc1[2] + acc1[3])
            ) * (1.0 / DEG)

        def t_body(t, carry):
            base = t * DEG
            pending = None
            for g in range(0, GROUPS, 2):
                sl0 = pl.ds(g * LANES, LANES)
                sl1 = pl.ds((g + 1) * LANES, LANES)
                acc0 = [rows_v[b, base + a, sl0] for a in range(4)]
                acc1 = [rows_v[b, base + a, sl1] for a in range(4)]
                if pending is not None:
                    flush(pending)
                for n in range(4, DEG, 4):
                    for a in range(4):
                        acc0[a] = acc0[a] + rows_v[b, base + n + a, sl0]
                    for a in range(4):
                        acc1[a] = acc1[a] + rows_v[b, base + n + a, sl1]
                pending = (t, sl0, sl1, acc0, acc1)
            flush(pending)
            return carry

        lax.fori_loop(0, CT, t_body, 0)

    for b in range(NBUF):
        start_gather(b, b)

    for b in range(NBUF):
        wait_gather(b, b)
        compute(b, b)
        start_out(b, b)
        start_gather(b + NBUF, b)

    def outer(i, carry):
        j = NBUF * i + NBUF
        for b in range(NBUF):
            jj = j + b
            wait_gather(jj, b)
            wait_out(jj - NBUF, b)
            compute(jj, b)
            start_out(jj, b)
            start_gather(jj + NBUF, b)
        return carry

    lax.fori_loop(0, NBLK - 2, outer, 0)

    for b in range(NBUF):
        jj = NCHUNK - NBUF + b
        wait_gather(jj, b)
        wait_out(jj - NBUF, b)
        compute(jj, b)
        start_out(jj, b)
    for b in range(NBUF):
        wait_out(NCHUNK - NBUF + b, b)


def kernel(neighbors, table):
    neigh = neighbors.astype(jnp.int32).reshape(NW, NCHUNK, ROWS_PER_DMA)
    mesh = plsc.VectorSubcoreMesh(core_axis_name="c", subcore_axis_name="s")
    k = pl.kernel(
        _body,
        mesh=mesh,
        out_type=jax.ShapeDtypeStruct((B, D), jnp.float32),
        scratch_types=[
            pltpu.VMEM((NCHUNK, ROWS_PER_DMA), jnp.int32),
            pltpu.VMEM((NBUF, ROWS_PER_DMA, D), jnp.float32),
            pltpu.VMEM((NBUF, CT, D), jnp.float32),
            pltpu.SemaphoreType.DMA((NBUF,)),
            pltpu.SemaphoreType.DMA((NBUF,)),
        ],
    )
    return k(neigh, table)


# speedup vs baseline: 1.3803x; 1.3803x over previous
import jax
import jax.numpy as jnp
from jax import lax
from jax.experimental import pallas as pl
from jax.experimental.pallas import tpu as pltpu
from jax.experimental.pallas import tpu_sc as plsc

B = 16384
DEG = 32
D = 128
NC = 2
NS = 16
NW = NC * NS
TPW = B // NW
ROWS_PER_IDX = 128
SPC = 2
CT = SPC * ROWS_PER_IDX // DEG
NCHUNK = TPW // CT
NBUF = 3
LANES = 16
GROUPS = D // LANES
NIDX = NCHUNK * SPC


def _body(neigh_hbm, table_hbm, out_hbm,
          idx_v, rows_v, outc_v, gsems, osems):
    wid = lax.axis_index("s") * NC + lax.axis_index("c")

    pltpu.sync_copy(neigh_hbm.at[wid], idx_v)

    def start_gather(chunk, b):
        for s in range(SPC):
            pltpu.async_copy(
                table_hbm.at[idx_v.at[chunk * SPC + s]],
                rows_v.at[b].at[pl.ds(s * ROWS_PER_IDX, ROWS_PER_IDX)],
                gsems.at[b])

    def wait_gather(chunk, b):
        for s in range(SPC):
            pltpu.make_async_copy(
                table_hbm.at[idx_v.at[chunk * SPC + s]],
                rows_v.at[b].at[pl.ds(s * ROWS_PER_IDX, ROWS_PER_IDX)],
                gsems.at[b]).wait()

    def out_slice(chunk):
        return out_hbm.at[pl.ds(wid * TPW + chunk * CT, CT)]

    def start_out(chunk, b):
        pltpu.async_copy(outc_v.at[b], out_slice(chunk), osems.at[b])

    def wait_out(chunk, b):
        pltpu.make_async_copy(outc_v.at[b], out_slice(chunk),
                              osems.at[b]).wait()

    def compute(chunk, b):
        def flush(p):
            t, sl0, sl1, acc0, acc1 = p
            outc_v[b, t, sl0] = (
                (acc0[0] + acc0[1]) + (acc0[2] + acc0[3])
            ) * (1.0 / DEG)
            outc_v[b, t, sl1] = (
                (acc1[0] + acc1[1]) + (acc1[2] + acc1[3])
            ) * (1.0 / DEG)

        def t_body(t, carry):
            base = t * DEG
            pending = None
            for g in range(0, GROUPS, 2):
                sl0 = pl.ds(g * LANES, LANES)
                sl1 = pl.ds((g + 1) * LANES, LANES)
                acc0 = [rows_v[b, base + a, sl0] for a in range(4)]
                acc1 = [rows_v[b, base + a, sl1] for a in range(4)]
                if pending is not None:
                    flush(pending)
                for n in range(4, DEG, 4):
                    for a in range(4):
                        acc0[a] = acc0[a] + rows_v[b, base + n + a, sl0]
                    for a in range(4):
                        acc1[a] = acc1[a] + rows_v[b, base + n + a, sl1]
                pending = (t, sl0, sl1, acc0, acc1)
            flush(pending)
            return carry

        lax.fori_loop(0, CT, t_body, 0)

    for b in range(NBUF):
        start_gather(b, b)

    for b in range(NBUF):
        wait_gather(b, b)
        compute(b, b)
        start_out(b, b)
        start_gather(b + NBUF, b)

    def outer(j, carry):
        b = lax.rem(j, NBUF)
        wait_gather(j, b)
        wait_out(j - NBUF, b)
        compute(j, b)
        start_out(j, b)
        start_gather(j + NBUF, b)
        return carry

    lax.fori_loop(NBUF, NCHUNK - NBUF, outer, 0)

    for b0 in range(NBUF):
        jj = NCHUNK - NBUF + b0
        b = jj % NBUF
        wait_gather(jj, b)
        wait_out(jj - NBUF, b)
        compute(jj, b)
        start_out(jj, b)
    for b0 in range(NBUF):
        jj = NCHUNK - NBUF + b0
        wait_out(jj, jj % NBUF)


def kernel(neighbors, table):
    neigh = neighbors.astype(jnp.int32).reshape(NW, NIDX, ROWS_PER_IDX)
    mesh = plsc.VectorSubcoreMesh(core_axis_name="c", subcore_axis_name="s")
    k = pl.kernel(
        _body,
        mesh=mesh,
        out_type=jax.ShapeDtypeStruct((B, D), jnp.float32),
        scratch_types=[
            pltpu.VMEM((NIDX, ROWS_PER_IDX), jnp.int32),
            pltpu.VMEM((NBUF, SPC * ROWS_PER_IDX, D), jnp.float32),
            pltpu.VMEM((NBUF, CT, D), jnp.float32),
            pltpu.SemaphoreType.DMA((NBUF,)),
            pltpu.SemaphoreType.DMA((NBUF,)),
        ],
    )
    return k(neigh, table)
